# Initial kernel scaffold; baseline (speedup 1.0000x reference)
import jax
import jax.numpy as jnp
from jax.experimental import pallas as pl


def kernel(adj_indices, adj_values, embeds):
    raise NotImplementedError("write your pallas kernel here")


# speedup vs baseline: 6.5239x; 6.5239x over previous
import functools

import jax
import jax.numpy as jnp
from jax import lax
from jax.experimental import pallas as pl
from jax.experimental.pallas import tpu as pltpu
from jax.experimental.pallas import tpu_sc as plsc

N_NODES = 10000
N_EDGES = 320000
D_FEAT = 128

NC = 2
NS = 16
NW = NC * NS

EPT = N_EDGES // NW
CHUNK = 80
NCHUNK = EPT // CHUNK
RPT = 624
TAIL_OFF = NS * RPT
TAIL = N_NODES - TAIL_OFF
VPF = D_FEAT // 16


def _sc_spmm(cols_hbm, rows_hbm, vals_hbm, embeds_hbm, zeros_hbm, out_hbm,
             cols_v, rows_v, vals_v, gath_v, acc_sh, sem):
    cid = lax.axis_index("c")
    sid = lax.axis_index("s")
    wid = cid * NS + sid

    pltpu.sync_copy(cols_hbm.at[wid], cols_v)
    pltpu.sync_copy(rows_hbm.at[wid], rows_v)
    pltpu.sync_copy(vals_hbm.at[wid], vals_v)

    pltpu.sync_copy(zeros_hbm.at[pl.ds(sid * RPT, RPT)],
                    acc_sh.at[pl.ds(sid * RPT, RPT)])

    @pl.when(sid == NS - 1)
    def _():
        pltpu.sync_copy(zeros_hbm.at[pl.ds(TAIL_OFF, TAIL)],
                        acc_sh.at[pl.ds(TAIL_OFF, TAIL)])

    plsc.subcore_barrier()

    def chunk_body(j, _):
        pltpu.async_copy(
            embeds_hbm.at[cols_v.at[pl.ds(j * CHUNK, CHUNK)]], gath_v, sem
        ).wait()

        for g in range(CHUNK // 16):
            vv16 = vals_v[pl.ds(j * CHUNK + g * 16, 16)]
            for i in range(16):
                e = g * 16 + i
                s = jnp.full((16,), vv16[i], jnp.float32)
                for k in range(VPF):
                    gath_v[e, pl.ds(k * 16, 16)] = (
                        gath_v[e, pl.ds(k * 16, 16)] * s)

        pltpu.sync_copy(gath_v, acc_sh.at[rows_v.at[j]], add=True)
        return 0

    lax.fori_loop(0, NCHUNK, chunk_body, 0)

    plsc.subcore_barrier()

    pltpu.sync_copy(acc_sh.at[pl.ds(sid * RPT, RPT)],
                    out_hbm.at[cid, pl.ds(sid * RPT, RPT)])

    @pl.when(sid == NS - 1)
    def _():
        pltpu.sync_copy(acc_sh.at[pl.ds(TAIL_OFF, TAIL)],
                        out_hbm.at[cid, pl.ds(TAIL_OFF, TAIL)])


def _combine(a_ref, b_ref, o_ref):
    o_ref[...] = a_ref[...] + b_ref[...]


@jax.jit
def kernel(adj_indices, adj_values, embeds):
    hbm = functools.partial(pltpu.with_memory_space_constraint,
                            memory_space=pltpu.MemorySpace.HBM)
    rows = hbm(adj_indices[0].reshape(NW, NCHUNK, CHUNK))
    cols = hbm(adj_indices[1].reshape(NW, EPT))
    vals = hbm(adj_values.reshape(NW, EPT))
    zeros = hbm(jnp.zeros((N_NODES, D_FEAT), jnp.float32))

    mesh = plsc.VectorSubcoreMesh(core_axis_name="c", subcore_axis_name="s")
    partials = pl.kernel(
        _sc_spmm,
        out_type=jax.ShapeDtypeStruct((NC, N_NODES, D_FEAT), jnp.float32),
        mesh=mesh,
        scratch_types=[
            pltpu.VMEM((EPT,), jnp.int32),
            pltpu.VMEM((NCHUNK, CHUNK), jnp.int32),
            pltpu.VMEM((EPT,), jnp.float32),
            pltpu.VMEM((CHUNK, D_FEAT), jnp.float32),
            pltpu.VMEM_SHARED((N_NODES, D_FEAT), jnp.float32),
            pltpu.SemaphoreType.DMA,
        ],
    )(cols, rows, vals, embeds, zeros)

    rows_blk = 1000
    out = pl.pallas_call(
        _combine,
        grid=(N_NODES // rows_blk,),
        in_specs=[pl.BlockSpec((rows_blk, D_FEAT), lambda i: (i, 0))] * 2,
        out_specs=pl.BlockSpec((rows_blk, D_FEAT), lambda i: (i, 0)),
        out_shape=jax.ShapeDtypeStruct((N_NODES, D_FEAT), jnp.float32),
    )(partials[0], partials[1])
    return out
